# baseline (speedup 1.0000x reference)
import functools

import jax
import jax.numpy as jnp
import numpy as np
from jax import lax
from jax.experimental import pallas as pl
from jax.experimental.pallas import tpu as pltpu
from jax.experimental.pallas import tpu_sc as plsc

N_LEVELS = 16
BASE_RES = 16
TABLE_SIZE = 2 ** 15

P1 = -1640531535
P2 = 805459861

CHUNK = 4096
BN = 4096


def _encode_body(tabs, xyz_h, bnds, out, tab_v, xyz_v,
                 f0_v, f1_v, f2_v, f3_v, m_v, b_v):
    c = lax.axis_index("c")
    s = lax.axis_index("s")
    n = xyz_h.shape[0] // 3
    half_n = n // 2
    is_mask_tile = s == 0

    pltpu.sync_copy(tabs.at[pl.ds(s * (2 * TABLE_SIZE), 2 * TABLE_SIZE)],
                    tab_v)
    pltpu.sync_copy(bnds, b_v)

    mnx = b_v[pl.ds(0, 16)]
    mny = b_v[pl.ds(16, 16)]
    mnz = b_v[pl.ds(32, 16)]
    rgx = b_v[pl.ds(48, 16)]
    rgy = b_v[pl.ds(64, 16)]
    rgz = b_v[pl.ds(80, 16)]

    res_i = lax.shift_left(jnp.int32(BASE_RES), s)
    resv = jnp.broadcast_to(res_i, (16,)).astype(jnp.float32)
    hi16 = jnp.int32(-65536)
    lane3 = lax.iota(jnp.int32, 16) * 3

    def chunk_body(k, _):
        base = c * half_n + k * CHUNK
        pltpu.sync_copy(xyz_h.at[pl.ds(3 * base, 3 * CHUNK)], xyz_v)

        def vec_body(i, _):
            off = i * 16
            i0 = lane3 + i * 48
            x01 = (plsc.load_gather(xyz_v, [i0]) - mnx) / rgx
            y01 = (plsc.load_gather(xyz_v, [i0 + 1]) - mny) / rgy
            z01 = (plsc.load_gather(xyz_v, [i0 + 2]) - mnz) / rgz
            px = x01 * resv
            py = y01 * resv
            pz = z01 * resv
            pxi = px.astype(jnp.int32)
            pyi = py.astype(jnp.int32)
            pzi = pz.astype(jnp.int32)
            fx = px - pxi.astype(jnp.float32)
            fy = py - pyi.astype(jnp.float32)
            fz = pz - pzi.astype(jnp.float32)
            gx = 1.0 - fx
            gy = 1.0 - fy
            gz = 1.0 - fz
            hx = [pxi, pxi + 1]
            hy0 = pyi * P1
            hy = [hy0, hy0 + P1]
            hz0 = pzi * P2
            hz = [hz0, hz0 + P2]
            wxy = [gx * gy, fx * gy, gx * fy, fx * fy]
            wz = [gz, fz]
            acc = [jnp.zeros((16,), jnp.float32) for _ in range(4)]
            for corner in range(8):
                bx = corner & 1
                by = (corner >> 1) & 1
                bz = corner >> 2
                h = hx[bx] ^ hy[by] ^ hz[bz]
                idx2 = lax.shift_left(h & (TABLE_SIZE - 1), 1)
                w0 = plsc.load_gather(tab_v, [idx2])
                w1 = plsc.load_gather(tab_v, [idx2 | 1])
                a0 = plsc.bitcast(lax.shift_left(w0, 16), jnp.float32)
                a1 = plsc.bitcast(w0 & hi16, jnp.float32)
                a2 = plsc.bitcast(lax.shift_left(w1, 16), jnp.float32)
                a3 = plsc.bitcast(w1 & hi16, jnp.float32)
                w = wxy[bx + 2 * by] * wz[bz]
                acc[0] = acc[0] + w * a0
                acc[1] = acc[1] + w * a1
                acc[2] = acc[2] + w * a2
                acc[3] = acc[3] + w * a3
            f0_v[pl.ds(off, 16)] = acc[0]
            f1_v[pl.ds(off, 16)] = acc[1]
            f2_v[pl.ds(off, 16)] = acc[2]
            f3_v[pl.ds(off, 16)] = acc[3]

            @pl.when(is_mask_tile)
            def _():
                inb = ((x01 >= 0.0) & (x01 <= 1.0)
                       & (y01 >= 0.0) & (y01 <= 1.0)
                       & (z01 >= 0.0) & (z01 <= 1.0))
                m_v[pl.ds(off, 16)] = jnp.where(inb, jnp.float32(1.0),
                                                jnp.float32(0.0))
            return None

        lax.fori_loop(0, CHUNK // 16, vec_body, None)

        r0 = 4 * s
        pltpu.sync_copy(f0_v, out.at[pl.ds(r0 * n + base, CHUNK)])
        pltpu.sync_copy(f1_v, out.at[pl.ds((r0 + 1) * n + base, CHUNK)])
        pltpu.sync_copy(f2_v, out.at[pl.ds((r0 + 2) * n + base, CHUNK)])
        pltpu.sync_copy(f3_v, out.at[pl.ds((r0 + 3) * n + base, CHUNK)])

        @pl.when(is_mask_tile)
        def _():
            pltpu.sync_copy(m_v, out.at[pl.ds(64 * n + base, CHUNK)])
        return None

    lax.fori_loop(0, half_n // CHUNK, chunk_body, None)


def _mlp_body(enc_ref, m_ref, w0_ref, w1_ref, w2_ref, o3_ref, o4_ref):
    enc = enc_ref[...]
    dn0 = (((0,), (0,)), ((), ()))
    dn1 = (((1,), (0,)), ((), ()))
    ht = jnp.maximum(
        lax.dot_general(enc, w0_ref[...], dn0,
                        preferred_element_type=jnp.float32), 0.0)
    ht = jnp.maximum(
        lax.dot_general(ht, w1_ref[...], dn1,
                        preferred_element_type=jnp.float32), 0.0)
    rt = lax.dot_general(ht, w2_ref[...], dn1,
                         preferred_element_type=jnp.float32)
    mcol = lax.dot_general(m_ref[...], jnp.ones((1, 1), jnp.float32), dn0,
                           preferred_element_type=jnp.float32)
    sel = mcol > 0.5
    o3_ref[...] = jnp.where(sel, rt[:, 0:3], 0.0)
    base4 = (lax.broadcasted_iota(jnp.int32, (1, 4), 1) == 0).astype(
        jnp.float32)
    o4_ref[...] = jnp.where(sel, rt[:, 3:7], base4)


def kernel(xyz, table, W0, W1, W2, xyz_bound_min, xyz_bound_max):
    n = xyz.shape[0]
    xyz_h = xyz.reshape(3 * n)
    tb = table.astype(jnp.bfloat16)
    tabs = jax.lax.bitcast_convert_type(
        tb.reshape(N_LEVELS, TABLE_SIZE, 2, 2), jnp.int32
    ).reshape(N_LEVELS * TABLE_SIZE * 2)
    rng = xyz_bound_max - xyz_bound_min
    bnds = jnp.concatenate([
        jnp.broadcast_to(xyz_bound_min[:, None], (3, 16)),
        jnp.broadcast_to(rng[:, None], (3, 16)),
    ]).reshape(96)

    mesh = plsc.VectorSubcoreMesh(core_axis_name="c", subcore_axis_name="s",
                                  num_cores=2, num_subcores=16)
    encode = functools.partial(
        pl.kernel,
        out_type=jax.ShapeDtypeStruct((72 * n,), jnp.float32),
        mesh=mesh,
        compiler_params=pltpu.CompilerParams(needs_layout_passes=False),
        scratch_types=[
            pltpu.VMEM((2 * TABLE_SIZE,), jnp.int32),
            pltpu.VMEM((3 * CHUNK,), jnp.float32),
            pltpu.VMEM((CHUNK,), jnp.float32),
            pltpu.VMEM((CHUNK,), jnp.float32),
            pltpu.VMEM((CHUNK,), jnp.float32),
            pltpu.VMEM((CHUNK,), jnp.float32),
            pltpu.VMEM((CHUNK,), jnp.float32),
            pltpu.VMEM((96,), jnp.float32),
        ],
    )(_encode_body)
    enc_flat = encode(tabs, xyz_h, bnds)
    enc = enc_flat[0:64 * n].reshape(64, n)
    mrow = enc_flat[64 * n:65 * n].reshape(1, n)

    d_xyz, d_rot = pl.pallas_call(
        _mlp_body,
        grid=(n // BN,),
        in_specs=[
            pl.BlockSpec((64, BN), lambda i: (0, i)),
            pl.BlockSpec((1, BN), lambda i: (0, i)),
            pl.BlockSpec((64, 64), lambda i: (0, 0)),
            pl.BlockSpec((64, 64), lambda i: (0, 0)),
            pl.BlockSpec((64, 8), lambda i: (0, 0)),
        ],
        out_specs=[
            pl.BlockSpec((BN, 3), lambda i: (i, 0)),
            pl.BlockSpec((BN, 4), lambda i: (i, 0)),
        ],
        out_shape=[
            jax.ShapeDtypeStruct((n, 3), jnp.float32),
            jax.ShapeDtypeStruct((n, 4), jnp.float32),
        ],
    )(enc, mrow, W0, W1, W2)

    mask = mrow.reshape(n) > 0.0
    return (mask, d_xyz, d_rot)


# speedup vs baseline: 1.1789x; 1.1789x over previous
import functools

import jax
import jax.numpy as jnp
import numpy as np
from jax import lax
from jax.experimental import pallas as pl
from jax.experimental.pallas import tpu as pltpu
from jax.experimental.pallas import tpu_sc as plsc

N_LEVELS = 16
BASE_RES = 16
TABLE_SIZE = 2 ** 15

P1 = -1640531535
P2 = 805459861

CHUNK = 4096
BN = 4096


def _encode_body(tabs, xyz_h, bnds, out, tab_v, xyz_v,
                 f0_v, f1_v, f2_v, f3_v, m_v, b_v):
    c = lax.axis_index("c")
    s = lax.axis_index("s")
    n = xyz_h.shape[0] // 3
    half_n = n // 2
    is_mask_tile = s == 0

    pltpu.sync_copy(tabs.at[pl.ds(s * (2 * TABLE_SIZE), 2 * TABLE_SIZE)],
                    tab_v)
    pltpu.sync_copy(bnds, b_v)

    mnx = b_v[pl.ds(0, 16)]
    mny = b_v[pl.ds(16, 16)]
    mnz = b_v[pl.ds(32, 16)]
    rgx = b_v[pl.ds(48, 16)]
    rgy = b_v[pl.ds(64, 16)]
    rgz = b_v[pl.ds(80, 16)]

    res_i = lax.shift_left(jnp.int32(BASE_RES), s)
    resv = jnp.broadcast_to(res_i, (16,)).astype(jnp.float32)
    hi16 = jnp.int32(-65536)
    lane3 = lax.iota(jnp.int32, 16) * 3

    def chunk_body(k, _):
        base = c * half_n + k * CHUNK
        pltpu.sync_copy(xyz_h.at[pl.ds(3 * base, 3 * CHUNK)], xyz_v)

        def vec_body(i, _):
            off = i * 16
            i0 = lane3 + i * 48
            x01 = (plsc.load_gather(xyz_v, [i0]) - mnx) / rgx
            y01 = (plsc.load_gather(xyz_v, [i0 + 1]) - mny) / rgy
            z01 = (plsc.load_gather(xyz_v, [i0 + 2]) - mnz) / rgz
            px = x01 * resv
            py = y01 * resv
            pz = z01 * resv
            pxi = px.astype(jnp.int32)
            pyi = py.astype(jnp.int32)
            pzi = pz.astype(jnp.int32)
            fx = px - pxi.astype(jnp.float32)
            fy = py - pyi.astype(jnp.float32)
            fz = pz - pzi.astype(jnp.float32)
            gx = 1.0 - fx
            gy = 1.0 - fy
            gz = 1.0 - fz
            hx = [pxi, pxi + 1]
            hy0 = pyi * P1
            hy = [hy0, hy0 + P1]
            hz0 = pzi * P2
            hz = [hz0, hz0 + P2]
            wxy = [gx * gy, fx * gy, gx * fy, fx * fy]
            wz = [gz, fz]
            acc = [jnp.zeros((16,), jnp.float32) for _ in range(4)]
            for corner in range(8):
                bx = corner & 1
                by = (corner >> 1) & 1
                bz = corner >> 2
                h = hx[bx] ^ hy[by] ^ hz[bz]
                idx2 = lax.shift_left(h & (TABLE_SIZE - 1), 1)
                w0 = plsc.load_gather(tab_v, [idx2])
                w1 = plsc.load_gather(tab_v, [idx2 | 1])
                a0 = plsc.bitcast(lax.shift_left(w0, 16), jnp.float32)
                a1 = plsc.bitcast(w0 & hi16, jnp.float32)
                a2 = plsc.bitcast(lax.shift_left(w1, 16), jnp.float32)
                a3 = plsc.bitcast(w1 & hi16, jnp.float32)
                w = wxy[bx + 2 * by] * wz[bz]
                acc[0] = acc[0] + w * a0
                acc[1] = acc[1] + w * a1
                acc[2] = acc[2] + w * a2
                acc[3] = acc[3] + w * a3
            f0_v[pl.ds(off, 16)] = acc[0]
            f1_v[pl.ds(off, 16)] = acc[1]
            f2_v[pl.ds(off, 16)] = acc[2]
            f3_v[pl.ds(off, 16)] = acc[3]

            @pl.when(is_mask_tile)
            def _():
                inb = ((x01 >= 0.0) & (x01 <= 1.0)
                       & (y01 >= 0.0) & (y01 <= 1.0)
                       & (z01 >= 0.0) & (z01 <= 1.0))
                m_v[pl.ds(off, 16)] = jnp.where(inb, jnp.float32(1.0),
                                                jnp.float32(0.0))
            return None

        lax.fori_loop(0, CHUNK // 16, vec_body, None)

        r0 = 4 * s
        pltpu.sync_copy(f0_v, out.at[pl.ds(r0 * n + base, CHUNK)])
        pltpu.sync_copy(f1_v, out.at[pl.ds((r0 + 1) * n + base, CHUNK)])
        pltpu.sync_copy(f2_v, out.at[pl.ds((r0 + 2) * n + base, CHUNK)])
        pltpu.sync_copy(f3_v, out.at[pl.ds((r0 + 3) * n + base, CHUNK)])

        @pl.when(is_mask_tile)
        def _():
            pltpu.sync_copy(m_v, out.at[pl.ds(64 * n + base, CHUNK)])
        return None

    lax.fori_loop(0, half_n // CHUNK, chunk_body, None)


def _mlp_body(enc_ref, m_ref, w0_ref, w1_ref, w2_ref, out_ref):
    enc = enc_ref[...]
    maskf = m_ref[...]
    dn = (((0,), (0,)), ((), ()))
    h = jnp.maximum(
        lax.dot_general(w0_ref[...], enc, dn,
                        preferred_element_type=jnp.float32), 0.0)
    h = jnp.maximum(
        lax.dot_general(w1_ref[...], h, dn,
                        preferred_element_type=jnp.float32), 0.0)
    r = lax.dot_general(w2_ref[...], h, dn,
                        preferred_element_type=jnp.float32)
    m = maskf > 0.5
    dxyz = jnp.where(m, r[0:3], 0.0)
    rot0 = jnp.where(m, r[3:4], 1.0)
    rot123 = jnp.where(m, r[4:7], 0.0)
    out_ref[...] = jnp.concatenate([dxyz, rot0, rot123, maskf], axis=0)


def kernel(xyz, table, W0, W1, W2, xyz_bound_min, xyz_bound_max):
    n = xyz.shape[0]
    xyz_h = xyz.reshape(3 * n)
    tb = table.astype(jnp.bfloat16)
    tabs = jax.lax.bitcast_convert_type(
        tb.reshape(N_LEVELS, TABLE_SIZE, 2, 2), jnp.int32
    ).reshape(N_LEVELS * TABLE_SIZE * 2)
    rng = xyz_bound_max - xyz_bound_min
    bnds = jnp.concatenate([
        jnp.broadcast_to(xyz_bound_min[:, None], (3, 16)),
        jnp.broadcast_to(rng[:, None], (3, 16)),
    ]).reshape(96)

    mesh = plsc.VectorSubcoreMesh(core_axis_name="c", subcore_axis_name="s",
                                  num_cores=2, num_subcores=16)
    encode = functools.partial(
        pl.kernel,
        out_type=jax.ShapeDtypeStruct((72 * n,), jnp.float32),
        mesh=mesh,
        compiler_params=pltpu.CompilerParams(needs_layout_passes=False),
        scratch_types=[
            pltpu.VMEM((2 * TABLE_SIZE,), jnp.int32),
            pltpu.VMEM((3 * CHUNK,), jnp.float32),
            pltpu.VMEM((CHUNK,), jnp.float32),
            pltpu.VMEM((CHUNK,), jnp.float32),
            pltpu.VMEM((CHUNK,), jnp.float32),
            pltpu.VMEM((CHUNK,), jnp.float32),
            pltpu.VMEM((CHUNK,), jnp.float32),
            pltpu.VMEM((96,), jnp.float32),
        ],
    )(_encode_body)
    enc_flat = encode(tabs, xyz_h, bnds)
    enc = enc_flat[0:64 * n].reshape(64, n)
    mrow = enc_flat[64 * n:65 * n].reshape(1, n)

    out8 = pl.pallas_call(
        _mlp_body,
        grid=(n // BN,),
        in_specs=[
            pl.BlockSpec((64, BN), lambda i: (0, i)),
            pl.BlockSpec((1, BN), lambda i: (0, i)),
            pl.BlockSpec((64, 64), lambda i: (0, 0)),
            pl.BlockSpec((64, 64), lambda i: (0, 0)),
            pl.BlockSpec((64, 8), lambda i: (0, 0)),
        ],
        out_specs=pl.BlockSpec((8, BN), lambda i: (0, i)),
        out_shape=jax.ShapeDtypeStruct((8, n), jnp.float32),
    )(enc, mrow, W0, W1, W2)

    mask = out8[7] > 0.0
    d_xyz = out8[0:3].T
    d_rot = out8[3:7].T
    return (mask, d_xyz, d_rot)


# speedup vs baseline: 1.2134x; 1.0293x over previous
import functools

import jax
import jax.numpy as jnp
import numpy as np
from jax import lax
from jax.experimental import pallas as pl
from jax.experimental.pallas import tpu as pltpu
from jax.experimental.pallas import tpu_sc as plsc

N_LEVELS = 16
BASE_RES = 16
TABLE_SIZE = 2 ** 15

P1 = -1640531535
P2 = 805459861

CHUNK = 4096
BN = 4096


def _encode_body(tabs, xyz_h, bnds, out, tab_v, xyz_v,
                 f0_v, f1_v, f2_v, f3_v, m_v, b_v):
    c = lax.axis_index("c")
    s = lax.axis_index("s")
    n = xyz_h.shape[0] // 3
    half_n = n // 2
    is_mask_tile = s == 0

    pltpu.sync_copy(tabs.at[pl.ds(s * (2 * TABLE_SIZE), 2 * TABLE_SIZE)],
                    tab_v)
    pltpu.sync_copy(bnds, b_v)

    mnx = b_v[pl.ds(0, 16)]
    mny = b_v[pl.ds(16, 16)]
    mnz = b_v[pl.ds(32, 16)]
    rgx = b_v[pl.ds(48, 16)]
    rgy = b_v[pl.ds(64, 16)]
    rgz = b_v[pl.ds(80, 16)]

    res_i = lax.shift_left(jnp.int32(BASE_RES), s)
    resv = jnp.broadcast_to(res_i, (16,)).astype(jnp.float32)
    hi16 = jnp.int32(-65536)
    lane3 = lax.iota(jnp.int32, 16) * 3

    def chunk_body(k, _):
        base = c * half_n + k * CHUNK
        pltpu.sync_copy(xyz_h.at[pl.ds(3 * base, 3 * CHUNK)], xyz_v)

        def vec_body(i, _):
            off = i * 16
            i0 = lane3 + i * 48
            x01 = (plsc.load_gather(xyz_v, [i0]) - mnx) / rgx
            y01 = (plsc.load_gather(xyz_v, [i0 + 1]) - mny) / rgy
            z01 = (plsc.load_gather(xyz_v, [i0 + 2]) - mnz) / rgz
            px = x01 * resv
            py = y01 * resv
            pz = z01 * resv
            pxi = px.astype(jnp.int32)
            pyi = py.astype(jnp.int32)
            pzi = pz.astype(jnp.int32)
            fx = px - pxi.astype(jnp.float32)
            fy = py - pyi.astype(jnp.float32)
            fz = pz - pzi.astype(jnp.float32)
            gx = 1.0 - fx
            gy = 1.0 - fy
            gz = 1.0 - fz
            hx = [pxi, pxi + 1]
            hy0 = pyi * P1
            hy = [hy0, hy0 + P1]
            hz0 = pzi * P2
            hz = [hz0, hz0 + P2]
            wxy = [gx * gy, fx * gy, gx * fy, fx * fy]
            wz = [gz, fz]
            acc = [jnp.zeros((16,), jnp.float32) for _ in range(4)]
            for corner in range(8):
                bx = corner & 1
                by = (corner >> 1) & 1
                bz = corner >> 2
                h = hx[bx] ^ hy[by] ^ hz[bz]
                idx2 = lax.shift_left(h & (TABLE_SIZE - 1), 1)
                w0 = plsc.load_gather(tab_v, [idx2])
                w1 = plsc.load_gather(tab_v, [idx2 | 1])
                a0 = plsc.bitcast(lax.shift_left(w0, 16), jnp.float32)
                a1 = plsc.bitcast(w0 & hi16, jnp.float32)
                a2 = plsc.bitcast(lax.shift_left(w1, 16), jnp.float32)
                a3 = plsc.bitcast(w1 & hi16, jnp.float32)
                w = wxy[bx + 2 * by] * wz[bz]
                acc[0] = acc[0] + w * a0
                acc[1] = acc[1] + w * a1
                acc[2] = acc[2] + w * a2
                acc[3] = acc[3] + w * a3
            f0_v[pl.ds(off, 16)] = acc[0]
            f1_v[pl.ds(off, 16)] = acc[1]
            f2_v[pl.ds(off, 16)] = acc[2]
            f3_v[pl.ds(off, 16)] = acc[3]

            @pl.when(is_mask_tile)
            def _():
                inb = ((x01 >= 0.0) & (x01 <= 1.0)
                       & (y01 >= 0.0) & (y01 <= 1.0)
                       & (z01 >= 0.0) & (z01 <= 1.0))
                m_v[pl.ds(off, 16)] = jnp.where(inb, jnp.float32(1.0),
                                                jnp.float32(0.0))
            return None

        lax.fori_loop(0, CHUNK // 16, vec_body, None)

        r0 = 4 * s
        pltpu.sync_copy(f0_v, out.at[pl.ds(r0 * n + base, CHUNK)])
        pltpu.sync_copy(f1_v, out.at[pl.ds((r0 + 1) * n + base, CHUNK)])
        pltpu.sync_copy(f2_v, out.at[pl.ds((r0 + 2) * n + base, CHUNK)])
        pltpu.sync_copy(f3_v, out.at[pl.ds((r0 + 3) * n + base, CHUNK)])

        @pl.when(is_mask_tile)
        def _():
            pltpu.sync_copy(m_v, out.at[pl.ds(64 * n + base, CHUNK)])
        return None

    lax.fori_loop(0, half_n // CHUNK, chunk_body, None)


def _mlp_body(enc_ref, w0_ref, w1_ref, w2_ref, out_ref):
    e = enc_ref[...]
    enc = e[0:64]
    maskf = e[64:65]
    dn = (((0,), (0,)), ((), ()))
    h = jnp.maximum(
        lax.dot_general(w0_ref[...], enc, dn,
                        preferred_element_type=jnp.float32), 0.0)
    h = jnp.maximum(
        lax.dot_general(w1_ref[...], h, dn,
                        preferred_element_type=jnp.float32), 0.0)
    r = lax.dot_general(w2_ref[...], h, dn,
                        preferred_element_type=jnp.float32)
    m = maskf > 0.5
    dxyz = jnp.where(m, r[0:3], 0.0)
    rot0 = jnp.where(m, r[3:4], 1.0)
    rot123 = jnp.where(m, r[4:7], 0.0)
    out_ref[...] = jnp.concatenate([dxyz, rot0, rot123, maskf], axis=0)


def kernel(xyz, table, W0, W1, W2, xyz_bound_min, xyz_bound_max):
    n = xyz.shape[0]
    xyz_h = xyz.reshape(3 * n)
    tb = table.astype(jnp.bfloat16)
    tabs = jax.lax.bitcast_convert_type(
        tb.reshape(N_LEVELS, TABLE_SIZE, 2, 2), jnp.int32
    ).reshape(N_LEVELS * TABLE_SIZE * 2)
    rng = xyz_bound_max - xyz_bound_min
    bnds = jnp.concatenate([
        jnp.broadcast_to(xyz_bound_min[:, None], (3, 16)),
        jnp.broadcast_to(rng[:, None], (3, 16)),
    ]).reshape(96)

    mesh = plsc.VectorSubcoreMesh(core_axis_name="c", subcore_axis_name="s",
                                  num_cores=2, num_subcores=16)
    encode = functools.partial(
        pl.kernel,
        out_type=jax.ShapeDtypeStruct((72 * n,), jnp.float32),
        mesh=mesh,
        compiler_params=pltpu.CompilerParams(needs_layout_passes=False),
        scratch_types=[
            pltpu.VMEM((2 * TABLE_SIZE,), jnp.int32),
            pltpu.VMEM((3 * CHUNK,), jnp.float32),
            pltpu.VMEM((CHUNK,), jnp.float32),
            pltpu.VMEM((CHUNK,), jnp.float32),
            pltpu.VMEM((CHUNK,), jnp.float32),
            pltpu.VMEM((CHUNK,), jnp.float32),
            pltpu.VMEM((CHUNK,), jnp.float32),
            pltpu.VMEM((96,), jnp.float32),
        ],
    )(_encode_body)
    enc = encode(tabs, xyz_h, bnds).reshape(72, n)

    out8 = pl.pallas_call(
        _mlp_body,
        grid=(n // BN,),
        in_specs=[
            pl.BlockSpec((72, BN), lambda i: (0, i)),
            pl.BlockSpec((64, 64), lambda i: (0, 0)),
            pl.BlockSpec((64, 64), lambda i: (0, 0)),
            pl.BlockSpec((64, 8), lambda i: (0, 0)),
        ],
        out_specs=pl.BlockSpec((8, BN), lambda i: (0, i)),
        out_shape=jax.ShapeDtypeStruct((8, n), jnp.float32),
    )(enc, W0, W1, W2)

    mask = out8[7] > 0.0
    d_xyz = out8[0:3].T
    d_rot = out8[3:7].T
    return (mask, d_xyz, d_rot)


# speedup vs baseline: 1.3233x; 1.0906x over previous
import functools

import jax
import jax.numpy as jnp
import numpy as np
from jax import lax
from jax.experimental import pallas as pl
from jax.experimental.pallas import tpu as pltpu
from jax.experimental.pallas import tpu_sc as plsc

N_LEVELS = 16
BASE_RES = 16
TABLE_SIZE = 2 ** 15

P1 = -1640531535
P2 = 805459861

CHUNK = 4096
BN = 4096


def _encode_body(tabs, x_h, y_h, z_h, bnds, out, tab_v, xyz_v,
                 f0_v, f1_v, f2_v, f3_v, m_v, b_v):
    c = lax.axis_index("c")
    s = lax.axis_index("s")
    n = x_h.shape[0]
    half_n = n // 2
    is_mask_tile = s == 0

    pltpu.sync_copy(tabs.at[pl.ds(s * (2 * TABLE_SIZE), 2 * TABLE_SIZE)],
                    tab_v)
    pltpu.sync_copy(bnds, b_v)

    mnx = b_v[pl.ds(0, 16)]
    mny = b_v[pl.ds(16, 16)]
    mnz = b_v[pl.ds(32, 16)]
    rgx = b_v[pl.ds(48, 16)]
    rgy = b_v[pl.ds(64, 16)]
    rgz = b_v[pl.ds(80, 16)]

    res_i = lax.shift_left(jnp.int32(BASE_RES), s)
    resv = jnp.broadcast_to(res_i, (16,)).astype(jnp.float32)
    hi16 = jnp.int32(-65536)

    def chunk_body(k, _):
        base = c * half_n + k * CHUNK
        pltpu.sync_copy(x_h.at[pl.ds(base, CHUNK)], xyz_v.at[pl.ds(0, CHUNK)])
        pltpu.sync_copy(y_h.at[pl.ds(base, CHUNK)],
                        xyz_v.at[pl.ds(CHUNK, CHUNK)])
        pltpu.sync_copy(z_h.at[pl.ds(base, CHUNK)],
                        xyz_v.at[pl.ds(2 * CHUNK, CHUNK)])

        def vec_body(i, _):
            off = i * 16
            x01 = (xyz_v[pl.ds(off, 16)] - mnx) / rgx
            y01 = (xyz_v[pl.ds(CHUNK + off, 16)] - mny) / rgy
            z01 = (xyz_v[pl.ds(2 * CHUNK + off, 16)] - mnz) / rgz
            px = x01 * resv
            py = y01 * resv
            pz = z01 * resv
            pxi = px.astype(jnp.int32)
            pyi = py.astype(jnp.int32)
            pzi = pz.astype(jnp.int32)
            fx = px - pxi.astype(jnp.float32)
            fy = py - pyi.astype(jnp.float32)
            fz = pz - pzi.astype(jnp.float32)
            gx = 1.0 - fx
            gy = 1.0 - fy
            gz = 1.0 - fz
            hx = [pxi, pxi + 1]
            hy0 = pyi * P1
            hy = [hy0, hy0 + P1]
            hz0 = pzi * P2
            hz = [hz0, hz0 + P2]
            wxy = [gx * gy, fx * gy, gx * fy, fx * fy]
            wz = [gz, fz]
            acc = [jnp.zeros((16,), jnp.float32) for _ in range(4)]
            for corner in range(8):
                bx = corner & 1
                by = (corner >> 1) & 1
                bz = corner >> 2
                h = hx[bx] ^ hy[by] ^ hz[bz]
                idx2 = lax.shift_left(h & (TABLE_SIZE - 1), 1)
                w0 = plsc.load_gather(tab_v, [idx2])
                w1 = plsc.load_gather(tab_v, [idx2 | 1])
                a0 = plsc.bitcast(lax.shift_left(w0, 16), jnp.float32)
                a1 = plsc.bitcast(w0 & hi16, jnp.float32)
                a2 = plsc.bitcast(lax.shift_left(w1, 16), jnp.float32)
                a3 = plsc.bitcast(w1 & hi16, jnp.float32)
                w = wxy[bx + 2 * by] * wz[bz]
                acc[0] = acc[0] + w * a0
                acc[1] = acc[1] + w * a1
                acc[2] = acc[2] + w * a2
                acc[3] = acc[3] + w * a3
            f0_v[pl.ds(off, 16)] = acc[0]
            f1_v[pl.ds(off, 16)] = acc[1]
            f2_v[pl.ds(off, 16)] = acc[2]
            f3_v[pl.ds(off, 16)] = acc[3]

            @pl.when(is_mask_tile)
            def _():
                inb = ((x01 >= 0.0) & (x01 <= 1.0)
                       & (y01 >= 0.0) & (y01 <= 1.0)
                       & (z01 >= 0.0) & (z01 <= 1.0))
                m_v[pl.ds(off, 16)] = jnp.where(inb, jnp.float32(1.0),
                                                jnp.float32(0.0))
            return None

        lax.fori_loop(0, CHUNK // 16, vec_body, None)

        r0 = 4 * s
        pltpu.sync_copy(f0_v, out.at[pl.ds(r0 * n + base, CHUNK)])
        pltpu.sync_copy(f1_v, out.at[pl.ds((r0 + 1) * n + base, CHUNK)])
        pltpu.sync_copy(f2_v, out.at[pl.ds((r0 + 2) * n + base, CHUNK)])
        pltpu.sync_copy(f3_v, out.at[pl.ds((r0 + 3) * n + base, CHUNK)])

        @pl.when(is_mask_tile)
        def _():
            pltpu.sync_copy(m_v, out.at[pl.ds(64 * n + base, CHUNK)])
        return None

    lax.fori_loop(0, half_n // CHUNK, chunk_body, None)


def _mlp_body(enc_ref, w0_ref, w1_ref, w2_ref, out_ref):
    e = enc_ref[...]
    enc = e[0:64]
    maskf = e[64:65]
    dn = (((0,), (0,)), ((), ()))
    h = jnp.maximum(
        lax.dot_general(w0_ref[...], enc, dn,
                        preferred_element_type=jnp.float32), 0.0)
    h = jnp.maximum(
        lax.dot_general(w1_ref[...], h, dn,
                        preferred_element_type=jnp.float32), 0.0)
    r = lax.dot_general(w2_ref[...], h, dn,
                        preferred_element_type=jnp.float32)
    m = maskf > 0.5
    dxyz = jnp.where(m, r[0:3], 0.0)
    rot0 = jnp.where(m, r[3:4], 1.0)
    rot123 = jnp.where(m, r[4:7], 0.0)
    out_ref[...] = jnp.concatenate([dxyz, rot0, rot123, maskf], axis=0)


def kernel(xyz, table, W0, W1, W2, xyz_bound_min, xyz_bound_max):
    n = xyz.shape[0]
    x_h = xyz[:, 0]
    y_h = xyz[:, 1]
    z_h = xyz[:, 2]
    tb = table.astype(jnp.bfloat16)
    tabs = jax.lax.bitcast_convert_type(
        tb.reshape(N_LEVELS, TABLE_SIZE, 2, 2), jnp.int32
    ).reshape(N_LEVELS * TABLE_SIZE * 2)
    rng = xyz_bound_max - xyz_bound_min
    bnds = jnp.concatenate([
        jnp.broadcast_to(xyz_bound_min[:, None], (3, 16)),
        jnp.broadcast_to(rng[:, None], (3, 16)),
    ]).reshape(96)

    mesh = plsc.VectorSubcoreMesh(core_axis_name="c", subcore_axis_name="s",
                                  num_cores=2, num_subcores=16)
    encode = functools.partial(
        pl.kernel,
        out_type=jax.ShapeDtypeStruct((72 * n,), jnp.float32),
        mesh=mesh,
        compiler_params=pltpu.CompilerParams(needs_layout_passes=False),
        scratch_types=[
            pltpu.VMEM((2 * TABLE_SIZE,), jnp.int32),
            pltpu.VMEM((3 * CHUNK,), jnp.float32),
            pltpu.VMEM((CHUNK,), jnp.float32),
            pltpu.VMEM((CHUNK,), jnp.float32),
            pltpu.VMEM((CHUNK,), jnp.float32),
            pltpu.VMEM((CHUNK,), jnp.float32),
            pltpu.VMEM((CHUNK,), jnp.float32),
            pltpu.VMEM((96,), jnp.float32),
        ],
    )(_encode_body)
    enc = encode(tabs, x_h, y_h, z_h, bnds).reshape(72, n)

    out8 = pl.pallas_call(
        _mlp_body,
        grid=(n // BN,),
        in_specs=[
            pl.BlockSpec((72, BN), lambda i: (0, i)),
            pl.BlockSpec((64, 64), lambda i: (0, 0)),
            pl.BlockSpec((64, 64), lambda i: (0, 0)),
            pl.BlockSpec((64, 8), lambda i: (0, 0)),
        ],
        out_specs=pl.BlockSpec((8, BN), lambda i: (0, i)),
        out_shape=jax.ShapeDtypeStruct((8, n), jnp.float32),
    )(enc, W0, W1, W2)

    mask = out8[7] > 0.0
    d_xyz = out8[0:3].T
    d_rot = out8[3:7].T
    return (mask, d_xyz, d_rot)


# speedup vs baseline: 1.5680x; 1.1849x over previous
import functools

import jax
import jax.numpy as jnp
import numpy as np
from jax import lax
from jax.experimental import pallas as pl
from jax.experimental.pallas import tpu as pltpu
from jax.experimental.pallas import tpu_sc as plsc

N_LEVELS = 16
BASE_RES = 16
TABLE_SIZE = 2 ** 15

P1 = -1640531535
P2 = 805459861

CHUNK = 4096
BN = 4096


def _encode_body(tabs, x_h, y_h, z_h, bnds, out, tab_v, xyz_v,
                 xc_v, yc_v, zc_v, idx_v,
                 f0_v, f1_v, f2_v, f3_v, m_v, b_v):
    c = lax.axis_index("c")
    s = lax.axis_index("s")
    n = x_h.shape[0]
    half_n = n // 2
    is_mask_tile = s == 0

    pltpu.sync_copy(tabs.at[pl.ds(s * (2 * TABLE_SIZE), 2 * TABLE_SIZE)],
                    tab_v)
    pltpu.sync_copy(bnds, b_v)

    mnx = b_v[pl.ds(0, 16)]
    mny = b_v[pl.ds(16, 16)]
    mnz = b_v[pl.ds(32, 16)]
    rgx = b_v[pl.ds(48, 16)]
    rgy = b_v[pl.ds(64, 16)]
    rgz = b_v[pl.ds(80, 16)]

    res_i = lax.shift_left(jnp.int32(BASE_RES), s)
    resv = jnp.broadcast_to(res_i, (16,)).astype(jnp.float32)
    hi16 = jnp.int32(-65536)
    lane16 = lax.iota(jnp.int32, 16)

    def chunk_body(k, _):
        base = c * half_n + k * CHUNK
        pltpu.sync_copy(x_h.at[pl.ds(base, CHUNK)], xyz_v.at[pl.ds(0, CHUNK)])
        pltpu.sync_copy(y_h.at[pl.ds(base, CHUNK)],
                        xyz_v.at[pl.ds(CHUNK, CHUNK)])
        pltpu.sync_copy(z_h.at[pl.ds(base, CHUNK)],
                        xyz_v.at[pl.ds(2 * CHUNK, CHUNK)])

        def p1_body(i, cnt):
            off = i * 16
            x01 = (xyz_v[pl.ds(off, 16)] - mnx) / rgx
            y01 = (xyz_v[pl.ds(CHUNK + off, 16)] - mny) / rgy
            z01 = (xyz_v[pl.ds(2 * CHUNK + off, 16)] - mnz) / rgz
            inb = ((x01 >= 0.0) & (x01 <= 1.0)
                   & (y01 >= 0.0) & (y01 <= 1.0)
                   & (z01 >= 0.0) & (z01 <= 1.0))
            plsc.store_compressed(xc_v.at[pl.ds(cnt, 16)], x01, mask=inb)
            plsc.store_compressed(yc_v.at[pl.ds(cnt, 16)], y01, mask=inb)
            plsc.store_compressed(zc_v.at[pl.ds(cnt, 16)], z01, mask=inb)
            plsc.store_compressed(idx_v.at[pl.ds(cnt, 16)], lane16 + off, mask=inb)

            @pl.when(is_mask_tile)
            def _():
                m_v[pl.ds(off, 16)] = jnp.where(inb, jnp.float32(1.0),
                                                jnp.float32(0.0))
            return cnt + jnp.sum(inb.astype(jnp.int32))

        cnt = lax.fori_loop(0, CHUNK // 16, p1_body, jnp.int32(0))
        idx_v[pl.ds(cnt, 16)] = lane16 + CHUNK

        def p2_body(j, _):
            off = j * 16
            x01 = xc_v[pl.ds(off, 16)]
            y01 = yc_v[pl.ds(off, 16)]
            z01 = zc_v[pl.ds(off, 16)]
            idxv = idx_v[pl.ds(off, 16)]
            px = x01 * resv
            py = y01 * resv
            pz = z01 * resv
            pxi = px.astype(jnp.int32)
            pyi = py.astype(jnp.int32)
            pzi = pz.astype(jnp.int32)
            fx = px - pxi.astype(jnp.float32)
            fy = py - pyi.astype(jnp.float32)
            fz = pz - pzi.astype(jnp.float32)
            gx = 1.0 - fx
            gy = 1.0 - fy
            gz = 1.0 - fz
            hx = [pxi, pxi + 1]
            hy0 = pyi * P1
            hy = [hy0, hy0 + P1]
            hz0 = pzi * P2
            hz = [hz0, hz0 + P2]
            wxy = [gx * gy, fx * gy, gx * fy, fx * fy]
            wz = [gz, fz]
            acc = [jnp.zeros((16,), jnp.float32) for _ in range(4)]
            for corner in range(8):
                bx = corner & 1
                by = (corner >> 1) & 1
                bz = corner >> 2
                h = hx[bx] ^ hy[by] ^ hz[bz]
                idx2 = lax.shift_left(h & (TABLE_SIZE - 1), 1)
                w0 = plsc.load_gather(tab_v, [idx2])
                w1 = plsc.load_gather(tab_v, [idx2 | 1])
                a0 = plsc.bitcast(lax.shift_left(w0, 16), jnp.float32)
                a1 = plsc.bitcast(w0 & hi16, jnp.float32)
                a2 = plsc.bitcast(lax.shift_left(w1, 16), jnp.float32)
                a3 = plsc.bitcast(w1 & hi16, jnp.float32)
                w = wxy[bx + 2 * by] * wz[bz]
                acc[0] = acc[0] + w * a0
                acc[1] = acc[1] + w * a1
                acc[2] = acc[2] + w * a2
                acc[3] = acc[3] + w * a3
            plsc.store_scatter(f0_v, [idxv], acc[0])
            plsc.store_scatter(f1_v, [idxv], acc[1])
            plsc.store_scatter(f2_v, [idxv], acc[2])
            plsc.store_scatter(f3_v, [idxv], acc[3])
            return None

        nv = lax.shift_right_logical(cnt + 15, 4)
        lax.fori_loop(0, nv, p2_body, None)

        r0 = 4 * s
        pltpu.sync_copy(f0_v.at[pl.ds(0, CHUNK)],
                        out.at[pl.ds(r0 * n + base, CHUNK)])
        pltpu.sync_copy(f1_v.at[pl.ds(0, CHUNK)],
                        out.at[pl.ds((r0 + 1) * n + base, CHUNK)])
        pltpu.sync_copy(f2_v.at[pl.ds(0, CHUNK)],
                        out.at[pl.ds((r0 + 2) * n + base, CHUNK)])
        pltpu.sync_copy(f3_v.at[pl.ds(0, CHUNK)],
                        out.at[pl.ds((r0 + 3) * n + base, CHUNK)])

        @pl.when(is_mask_tile)
        def _():
            pltpu.sync_copy(m_v, out.at[pl.ds(64 * n + base, CHUNK)])
        return None

    lax.fori_loop(0, half_n // CHUNK, chunk_body, None)


def _mlp_body(enc_ref, w0_ref, w1_ref, w2_ref, out_ref):
    e = enc_ref[...]
    enc = e[0:64]
    maskf = e[64:65]
    dn = (((0,), (0,)), ((), ()))
    h = jnp.maximum(
        lax.dot_general(w0_ref[...], enc, dn,
                        preferred_element_type=jnp.float32), 0.0)
    h = jnp.maximum(
        lax.dot_general(w1_ref[...], h, dn,
                        preferred_element_type=jnp.float32), 0.0)
    r = lax.dot_general(w2_ref[...], h, dn,
                        preferred_element_type=jnp.float32)
    m = maskf > 0.5
    dxyz = jnp.where(m, r[0:3], 0.0)
    rot0 = jnp.where(m, r[3:4], 1.0)
    rot123 = jnp.where(m, r[4:7], 0.0)
    out_ref[...] = jnp.concatenate([dxyz, rot0, rot123, maskf], axis=0)


def kernel(xyz, table, W0, W1, W2, xyz_bound_min, xyz_bound_max):
    n = xyz.shape[0]
    x_h = xyz[:, 0]
    y_h = xyz[:, 1]
    z_h = xyz[:, 2]
    tb = table.astype(jnp.bfloat16)
    tabs = jax.lax.bitcast_convert_type(
        tb.reshape(N_LEVELS, TABLE_SIZE, 2, 2), jnp.int32
    ).reshape(N_LEVELS * TABLE_SIZE * 2)
    rng = xyz_bound_max - xyz_bound_min
    bnds = jnp.concatenate([
        jnp.broadcast_to(xyz_bound_min[:, None], (3, 16)),
        jnp.broadcast_to(rng[:, None], (3, 16)),
    ]).reshape(96)

    mesh = plsc.VectorSubcoreMesh(core_axis_name="c", subcore_axis_name="s",
                                  num_cores=2, num_subcores=16)
    encode = functools.partial(
        pl.kernel,
        out_type=jax.ShapeDtypeStruct((72 * n,), jnp.float32),
        mesh=mesh,
        compiler_params=pltpu.CompilerParams(needs_layout_passes=False),
        scratch_types=[
            pltpu.VMEM((2 * TABLE_SIZE,), jnp.int32),
            pltpu.VMEM((3 * CHUNK,), jnp.float32),
            pltpu.VMEM((CHUNK + 16,), jnp.float32),
            pltpu.VMEM((CHUNK + 16,), jnp.float32),
            pltpu.VMEM((CHUNK + 16,), jnp.float32),
            pltpu.VMEM((CHUNK + 16,), jnp.int32),
            pltpu.VMEM((CHUNK + 16,), jnp.float32),
            pltpu.VMEM((CHUNK + 16,), jnp.float32),
            pltpu.VMEM((CHUNK + 16,), jnp.float32),
            pltpu.VMEM((CHUNK + 16,), jnp.float32),
            pltpu.VMEM((CHUNK,), jnp.float32),
            pltpu.VMEM((96,), jnp.float32),
        ],
    )(_encode_body)
    enc = encode(tabs, x_h, y_h, z_h, bnds).reshape(72, n)

    out8 = pl.pallas_call(
        _mlp_body,
        grid=(n // BN,),
        in_specs=[
            pl.BlockSpec((72, BN), lambda i: (0, i)),
            pl.BlockSpec((64, 64), lambda i: (0, 0)),
            pl.BlockSpec((64, 64), lambda i: (0, 0)),
            pl.BlockSpec((64, 8), lambda i: (0, 0)),
        ],
        out_specs=pl.BlockSpec((8, BN), lambda i: (0, i)),
        out_shape=jax.ShapeDtypeStruct((8, n), jnp.float32),
    )(enc, W0, W1, W2)

    mask = out8[7] > 0.0
    d_xyz = out8[0:3].T
    d_rot = out8[3:7].T
    return (mask, d_xyz, d_rot)


# speedup vs baseline: 1.5697x; 1.0011x over previous
import functools

import jax
import jax.numpy as jnp
import numpy as np
from jax import lax
from jax.experimental import pallas as pl
from jax.experimental.pallas import tpu as pltpu
from jax.experimental.pallas import tpu_sc as plsc

N_LEVELS = 16
BASE_RES = 16
TABLE_SIZE = 2 ** 15

P1 = -1640531535
P2 = 805459861

CHUNK = 4096
BN = 4096


def _encode_body(tabs, xt_h, bnds, out, tab_v, xyz_v,
                 xc_v, yc_v, zc_v, idx_v,
                 f0_v, f1_v, f2_v, f3_v, m_v, b_v):
    c = lax.axis_index("c")
    s = lax.axis_index("s")
    n = xt_h.shape[0] // 3
    half_n = n // 2
    is_mask_tile = s == 0

    pltpu.sync_copy(tabs.at[pl.ds(s * (2 * TABLE_SIZE), 2 * TABLE_SIZE)],
                    tab_v)
    pltpu.sync_copy(bnds, b_v)

    mnx = b_v[pl.ds(0, 16)]
    mny = b_v[pl.ds(16, 16)]
    mnz = b_v[pl.ds(32, 16)]
    rgx = b_v[pl.ds(48, 16)]
    rgy = b_v[pl.ds(64, 16)]
    rgz = b_v[pl.ds(80, 16)]

    res_i = lax.shift_left(jnp.int32(BASE_RES), s)
    resv = jnp.broadcast_to(res_i, (16,)).astype(jnp.float32)
    hi16 = jnp.int32(-65536)
    lane16 = lax.iota(jnp.int32, 16)

    def chunk_body(k, _):
        base = c * half_n + k * CHUNK
        pltpu.sync_copy(xt_h.at[pl.ds(base, CHUNK)],
                        xyz_v.at[pl.ds(0, CHUNK)])
        pltpu.sync_copy(xt_h.at[pl.ds(n + base, CHUNK)],
                        xyz_v.at[pl.ds(CHUNK, CHUNK)])
        pltpu.sync_copy(xt_h.at[pl.ds(2 * n + base, CHUNK)],
                        xyz_v.at[pl.ds(2 * CHUNK, CHUNK)])

        def p1_body(i, cnt):
            off = i * 16
            x01 = (xyz_v[pl.ds(off, 16)] - mnx) / rgx
            y01 = (xyz_v[pl.ds(CHUNK + off, 16)] - mny) / rgy
            z01 = (xyz_v[pl.ds(2 * CHUNK + off, 16)] - mnz) / rgz
            inb = ((x01 >= 0.0) & (x01 <= 1.0)
                   & (y01 >= 0.0) & (y01 <= 1.0)
                   & (z01 >= 0.0) & (z01 <= 1.0))
            plsc.store_compressed(xc_v.at[pl.ds(cnt, 16)], x01, mask=inb)
            plsc.store_compressed(yc_v.at[pl.ds(cnt, 16)], y01, mask=inb)
            plsc.store_compressed(zc_v.at[pl.ds(cnt, 16)], z01, mask=inb)
            plsc.store_compressed(idx_v.at[pl.ds(cnt, 16)], lane16 + off, mask=inb)

            @pl.when(is_mask_tile)
            def _():
                m_v[pl.ds(off, 16)] = jnp.where(inb, jnp.float32(1.0),
                                                jnp.float32(0.0))
            return cnt + jnp.sum(inb.astype(jnp.int32))

        cnt = lax.fori_loop(0, CHUNK // 16, p1_body, jnp.int32(0))
        idx_v[pl.ds(cnt, 16)] = lane16 + CHUNK

        def p2_body(j, _):
            off = j * 16
            x01 = xc_v[pl.ds(off, 16)]
            y01 = yc_v[pl.ds(off, 16)]
            z01 = zc_v[pl.ds(off, 16)]
            idxv = idx_v[pl.ds(off, 16)]
            px = x01 * resv
            py = y01 * resv
            pz = z01 * resv
            pxi = px.astype(jnp.int32)
            pyi = py.astype(jnp.int32)
            pzi = pz.astype(jnp.int32)
            fx = px - pxi.astype(jnp.float32)
            fy = py - pyi.astype(jnp.float32)
            fz = pz - pzi.astype(jnp.float32)
            gx = 1.0 - fx
            gy = 1.0 - fy
            gz = 1.0 - fz
            hx = [pxi, pxi + 1]
            hy0 = pyi * P1
            hy = [hy0, hy0 + P1]
            hz0 = pzi * P2
            hz = [hz0, hz0 + P2]
            wxy = [gx * gy, fx * gy, gx * fy, fx * fy]
            wz = [gz, fz]
            acc = [jnp.zeros((16,), jnp.float32) for _ in range(4)]
            for corner in range(8):
                bx = corner & 1
                by = (corner >> 1) & 1
                bz = corner >> 2
                h = hx[bx] ^ hy[by] ^ hz[bz]
                idx2 = lax.shift_left(h & (TABLE_SIZE - 1), 1)
                w0 = plsc.load_gather(tab_v, [idx2])
                w1 = plsc.load_gather(tab_v, [idx2 | 1])
                a0 = plsc.bitcast(lax.shift_left(w0, 16), jnp.float32)
                a1 = plsc.bitcast(w0 & hi16, jnp.float32)
                a2 = plsc.bitcast(lax.shift_left(w1, 16), jnp.float32)
                a3 = plsc.bitcast(w1 & hi16, jnp.float32)
                w = wxy[bx + 2 * by] * wz[bz]
                acc[0] = acc[0] + w * a0
                acc[1] = acc[1] + w * a1
                acc[2] = acc[2] + w * a2
                acc[3] = acc[3] + w * a3
            plsc.store_scatter(f0_v, [idxv], acc[0])
            plsc.store_scatter(f1_v, [idxv], acc[1])
            plsc.store_scatter(f2_v, [idxv], acc[2])
            plsc.store_scatter(f3_v, [idxv], acc[3])
            return None

        nv = lax.shift_right_logical(cnt + 15, 4)
        lax.fori_loop(0, nv, p2_body, None)

        r0 = 4 * s
        pltpu.sync_copy(f0_v.at[pl.ds(0, CHUNK)],
                        out.at[pl.ds(r0 * n + base, CHUNK)])
        pltpu.sync_copy(f1_v.at[pl.ds(0, CHUNK)],
                        out.at[pl.ds((r0 + 1) * n + base, CHUNK)])
        pltpu.sync_copy(f2_v.at[pl.ds(0, CHUNK)],
                        out.at[pl.ds((r0 + 2) * n + base, CHUNK)])
        pltpu.sync_copy(f3_v.at[pl.ds(0, CHUNK)],
                        out.at[pl.ds((r0 + 3) * n + base, CHUNK)])

        @pl.when(is_mask_tile)
        def _():
            pltpu.sync_copy(m_v, out.at[pl.ds(64 * n + base, CHUNK)])
        return None

    lax.fori_loop(0, half_n // CHUNK, chunk_body, None)


def _mlp_body(enc_ref, w0_ref, w1_ref, w2_ref, out_ref):
    e = enc_ref[...]
    enc = e[0:64]
    maskf = e[64:65]
    dn = (((0,), (0,)), ((), ()))
    h = jnp.maximum(
        lax.dot_general(w0_ref[...], enc, dn,
                        preferred_element_type=jnp.float32), 0.0)
    h = jnp.maximum(
        lax.dot_general(w1_ref[...], h, dn,
                        preferred_element_type=jnp.float32), 0.0)
    r = lax.dot_general(w2_ref[...], h, dn,
                        preferred_element_type=jnp.float32)
    m = maskf > 0.5
    dxyz = jnp.where(m, r[0:3], 0.0)
    rot0 = jnp.where(m, r[3:4], 1.0)
    rot123 = jnp.where(m, r[4:7], 0.0)
    out_ref[...] = jnp.concatenate([dxyz, rot0, rot123, maskf], axis=0)


def kernel(xyz, table, W0, W1, W2, xyz_bound_min, xyz_bound_max):
    n = xyz.shape[0]
    xt_h = xyz.T.reshape(3 * n)
    tb = table.astype(jnp.bfloat16)
    tabs = jax.lax.bitcast_convert_type(
        tb.reshape(N_LEVELS, TABLE_SIZE, 2, 2), jnp.int32
    ).reshape(N_LEVELS * TABLE_SIZE * 2)
    rng = xyz_bound_max - xyz_bound_min
    bnds = jnp.concatenate([
        jnp.broadcast_to(xyz_bound_min[:, None], (3, 16)),
        jnp.broadcast_to(rng[:, None], (3, 16)),
    ]).reshape(96)

    mesh = plsc.VectorSubcoreMesh(core_axis_name="c", subcore_axis_name="s",
                                  num_cores=2, num_subcores=16)
    encode = functools.partial(
        pl.kernel,
        out_type=jax.ShapeDtypeStruct((72 * n,), jnp.float32),
        mesh=mesh,
        compiler_params=pltpu.CompilerParams(needs_layout_passes=False),
        scratch_types=[
            pltpu.VMEM((2 * TABLE_SIZE,), jnp.int32),
            pltpu.VMEM((3 * CHUNK,), jnp.float32),
            pltpu.VMEM((CHUNK + 16,), jnp.float32),
            pltpu.VMEM((CHUNK + 16,), jnp.float32),
            pltpu.VMEM((CHUNK + 16,), jnp.float32),
            pltpu.VMEM((CHUNK + 16,), jnp.int32),
            pltpu.VMEM((CHUNK + 16,), jnp.float32),
            pltpu.VMEM((CHUNK + 16,), jnp.float32),
            pltpu.VMEM((CHUNK + 16,), jnp.float32),
            pltpu.VMEM((CHUNK + 16,), jnp.float32),
            pltpu.VMEM((CHUNK,), jnp.float32),
            pltpu.VMEM((96,), jnp.float32),
        ],
    )(_encode_body)
    enc = encode(tabs, xt_h, bnds).reshape(72, n)

    out8 = pl.pallas_call(
        _mlp_body,
        grid=(n // BN,),
        in_specs=[
            pl.BlockSpec((72, BN), lambda i: (0, i)),
            pl.BlockSpec((64, 64), lambda i: (0, 0)),
            pl.BlockSpec((64, 64), lambda i: (0, 0)),
            pl.BlockSpec((64, 8), lambda i: (0, 0)),
        ],
        out_specs=pl.BlockSpec((8, BN), lambda i: (0, i)),
        out_shape=jax.ShapeDtypeStruct((8, n), jnp.float32),
    )(enc, W0, W1, W2)

    mask = out8[7] > 0.0
    d_xyz = out8[0:3].T
    d_rot = out8[3:7].T
    return (mask, d_xyz, d_rot)


# speedup vs baseline: 1.6699x; 1.0638x over previous
import functools

import jax
import jax.numpy as jnp
import numpy as np
from jax import lax
from jax.experimental import pallas as pl
from jax.experimental.pallas import tpu as pltpu
from jax.experimental.pallas import tpu_sc as plsc

N_LEVELS = 16
BASE_RES = 16
TABLE_SIZE = 2 ** 15

P1 = -1640531535
P2 = 805459861

CHUNK = 4096
BN = 4096


def _encode_body(tabs, xt_h, bnds, out, tab_v, xyz_v,
                 xc_v, yc_v, zc_v, idx_v, f_v, m_v, b_v):
    c = lax.axis_index("c")
    s = lax.axis_index("s")
    n = xt_h.shape[0] // 3
    half_n = n // 2
    is_mask_tile = s == 0

    pltpu.sync_copy(tabs.at[pl.ds(s * (2 * TABLE_SIZE), 2 * TABLE_SIZE)],
                    tab_v)
    pltpu.sync_copy(bnds, b_v)

    mnx = b_v[pl.ds(0, 16)]
    mny = b_v[pl.ds(16, 16)]
    mnz = b_v[pl.ds(32, 16)]
    rgx = b_v[pl.ds(48, 16)]
    rgy = b_v[pl.ds(64, 16)]
    rgz = b_v[pl.ds(80, 16)]

    res_i = lax.shift_left(jnp.int32(BASE_RES), s)
    resv = jnp.broadcast_to(res_i, (16,)).astype(jnp.float32)
    hi16 = jnp.int32(-65536)
    lane16 = lax.iota(jnp.int32, 16)

    def chunk_body(k, _):
        g = c * (half_n // CHUNK) + k
        pltpu.sync_copy(xt_h.at[pl.ds(g * (3 * CHUNK), 3 * CHUNK)], xyz_v)

        def p1_body(i, cnt):
            off = i * 16
            x01 = (xyz_v[pl.ds(off, 16)] - mnx) / rgx
            y01 = (xyz_v[pl.ds(CHUNK + off, 16)] - mny) / rgy
            z01 = (xyz_v[pl.ds(2 * CHUNK + off, 16)] - mnz) / rgz
            inb = ((x01 >= 0.0) & (x01 <= 1.0)
                   & (y01 >= 0.0) & (y01 <= 1.0)
                   & (z01 >= 0.0) & (z01 <= 1.0))
            plsc.store_compressed(xc_v.at[pl.ds(cnt, 16)], x01, mask=inb)
            plsc.store_compressed(yc_v.at[pl.ds(cnt, 16)], y01, mask=inb)
            plsc.store_compressed(zc_v.at[pl.ds(cnt, 16)], z01, mask=inb)
            plsc.store_compressed(idx_v.at[pl.ds(cnt, 16)], lane16 + off, mask=inb)

            @pl.when(is_mask_tile)
            def _():
                m_v[pl.ds(off, 16)] = jnp.where(inb, jnp.float32(1.0),
                                                jnp.float32(0.0))
            return cnt + jnp.sum(inb.astype(jnp.int32))

        cnt = lax.fori_loop(0, CHUNK // 16, p1_body, jnp.int32(0))
        idx_v[pl.ds(cnt, 16)] = lane16 + 4 * CHUNK

        def p2_body(j, _):
            off = j * 16
            x01 = xc_v[pl.ds(off, 16)]
            y01 = yc_v[pl.ds(off, 16)]
            z01 = zc_v[pl.ds(off, 16)]
            idxv = idx_v[pl.ds(off, 16)]
            px = x01 * resv
            py = y01 * resv
            pz = z01 * resv
            pxi = px.astype(jnp.int32)
            pyi = py.astype(jnp.int32)
            pzi = pz.astype(jnp.int32)
            fx = px - pxi.astype(jnp.float32)
            fy = py - pyi.astype(jnp.float32)
            fz = pz - pzi.astype(jnp.float32)
            gx = 1.0 - fx
            gy = 1.0 - fy
            gz = 1.0 - fz
            hx = [pxi, pxi + 1]
            hy0 = pyi * P1
            hy = [hy0, hy0 + P1]
            hz0 = pzi * P2
            hz = [hz0, hz0 + P2]
            wxy = [gx * gy, fx * gy, gx * fy, fx * fy]
            wz = [gz, fz]
            acc = [jnp.zeros((16,), jnp.float32) for _ in range(4)]
            for corner in range(8):
                bx = corner & 1
                by = (corner >> 1) & 1
                bz = corner >> 2
                h = hx[bx] ^ hy[by] ^ hz[bz]
                idx2 = lax.shift_left(h & (TABLE_SIZE - 1), 1)
                w0 = plsc.load_gather(tab_v, [idx2])
                w1 = plsc.load_gather(tab_v, [idx2 | 1])
                a0 = plsc.bitcast(lax.shift_left(w0, 16), jnp.float32)
                a1 = plsc.bitcast(w0 & hi16, jnp.float32)
                a2 = plsc.bitcast(lax.shift_left(w1, 16), jnp.float32)
                a3 = plsc.bitcast(w1 & hi16, jnp.float32)
                w = wxy[bx + 2 * by] * wz[bz]
                acc[0] = acc[0] + w * a0
                acc[1] = acc[1] + w * a1
                acc[2] = acc[2] + w * a2
                acc[3] = acc[3] + w * a3
            plsc.store_scatter(f_v, [idxv], acc[0])
            plsc.store_scatter(f_v, [idxv + CHUNK], acc[1])
            plsc.store_scatter(f_v, [idxv + 2 * CHUNK], acc[2])
            plsc.store_scatter(f_v, [idxv + 3 * CHUNK], acc[3])
            return None

        nv = lax.shift_right_logical(cnt + 15, 4)
        lax.fori_loop(0, nv, p2_body, None)

        slab = g * (72 * CHUNK)
        pltpu.sync_copy(f_v.at[pl.ds(0, 4 * CHUNK)],
                        out.at[pl.ds(slab + 4 * s * CHUNK, 4 * CHUNK)])

        @pl.when(is_mask_tile)
        def _():
            pltpu.sync_copy(m_v, out.at[pl.ds(slab + 64 * CHUNK, CHUNK)])
        return None

    lax.fori_loop(0, half_n // CHUNK, chunk_body, None)


def _mlp_body(enc_ref, w0_ref, w1_ref, w2_ref, out_ref):
    e = enc_ref[...]
    enc = e[0:64]
    maskf = e[64:65]
    dn = (((0,), (0,)), ((), ()))
    h = jnp.maximum(
        lax.dot_general(w0_ref[...], enc, dn,
                        preferred_element_type=jnp.float32), 0.0)
    h = jnp.maximum(
        lax.dot_general(w1_ref[...], h, dn,
                        preferred_element_type=jnp.float32), 0.0)
    r = lax.dot_general(w2_ref[...], h, dn,
                        preferred_element_type=jnp.float32)
    m = maskf > 0.5
    dxyz = jnp.where(m, r[0:3], 0.0)
    rot0 = jnp.where(m, r[3:4], 1.0)
    rot123 = jnp.where(m, r[4:7], 0.0)
    out_ref[...] = jnp.concatenate([dxyz, rot0, rot123, maskf], axis=0)


def kernel(xyz, table, W0, W1, W2, xyz_bound_min, xyz_bound_max):
    n = xyz.shape[0]
    xt_h = (xyz.T.reshape(3, n // CHUNK, CHUNK)
            .transpose(1, 0, 2).reshape(3 * n))
    tb = table.astype(jnp.bfloat16)
    tabs = jax.lax.bitcast_convert_type(
        tb.reshape(N_LEVELS, TABLE_SIZE, 2, 2), jnp.int32
    ).reshape(N_LEVELS * TABLE_SIZE * 2)
    rng = xyz_bound_max - xyz_bound_min
    bnds = jnp.concatenate([
        jnp.broadcast_to(xyz_bound_min[:, None], (3, 16)),
        jnp.broadcast_to(rng[:, None], (3, 16)),
    ]).reshape(96)

    mesh = plsc.VectorSubcoreMesh(core_axis_name="c", subcore_axis_name="s",
                                  num_cores=2, num_subcores=16)
    encode = functools.partial(
        pl.kernel,
        out_type=jax.ShapeDtypeStruct((72 * n,), jnp.float32),
        mesh=mesh,
        compiler_params=pltpu.CompilerParams(needs_layout_passes=False),
        scratch_types=[
            pltpu.VMEM((2 * TABLE_SIZE,), jnp.int32),
            pltpu.VMEM((3 * CHUNK,), jnp.float32),
            pltpu.VMEM((CHUNK + 16,), jnp.float32),
            pltpu.VMEM((CHUNK + 16,), jnp.float32),
            pltpu.VMEM((CHUNK + 16,), jnp.float32),
            pltpu.VMEM((CHUNK + 16,), jnp.int32),
            pltpu.VMEM((7 * CHUNK + 16,), jnp.float32),
            pltpu.VMEM((CHUNK,), jnp.float32),
            pltpu.VMEM((96,), jnp.float32),
        ],
    )(_encode_body)
    enc = encode(tabs, xt_h, bnds).reshape((n // CHUNK) * 72, CHUNK)

    out8 = pl.pallas_call(
        _mlp_body,
        grid=(n // BN,),
        in_specs=[
            pl.BlockSpec((72, BN), lambda i: (i, 0)),
            pl.BlockSpec((64, 64), lambda i: (0, 0)),
            pl.BlockSpec((64, 64), lambda i: (0, 0)),
            pl.BlockSpec((64, 8), lambda i: (0, 0)),
        ],
        out_specs=pl.BlockSpec((8, BN), lambda i: (0, i)),
        out_shape=jax.ShapeDtypeStruct((8, n), jnp.float32),
    )(enc, W0, W1, W2)

    mask = out8[7] > 0.0
    d_xyz = out8[0:3].T
    d_rot = out8[3:7].T
    return (mask, d_xyz, d_rot)
